# Initial kernel scaffold; baseline (speedup 1.0000x reference)
import jax
import jax.numpy as jnp
from jax.experimental import pallas as pl


def kernel(type_ids, rel0_values, rel1_values, rel2_values, init_random, r0_W1, r0_b1, r0_W2, r0_b2, r1_W1, r1_b1, r1_W2, r1_b2, r2_W1, r2_b1, r2_W2, r2_b2, u_W1, u_b1, u_W2, u_b2, v_W1, v_b1, v_W2, v_b2):
    raise NotImplementedError("write your pallas kernel here")


# speedup vs baseline: 2.0903x; 2.0903x over previous
import functools

import jax
import jax.numpy as jnp
from jax import lax
from jax.experimental import pallas as pl
from jax.experimental.pallas import tpu as pltpu
from jax.experimental.pallas import tpu_sc as plsc

N = 10000
T = 8
H = 128
MAXOBJ = 20000

NC = 2
NS = 16
NW = NC * NS

B01 = 128
C01 = 320000 // B01
B2 = 80
C2 = 10000 // B2
K01 = -(-C01 // NW)
K2 = -(-C2 // NW)
ZB = 80
CZ = N // ZB
KZ = -(-CZ // NS)

@functools.cache
def _build_gather():
  mesh = plsc.VectorSubcoreMesh(core_axis_name="c", subcore_axis_name="s",
                                num_cores=NC, num_subcores=NS)

  @functools.partial(
      pl.kernel,
      out_type=(jax.ShapeDtypeStruct((C01 * B01, H), jnp.float32),
                jax.ShapeDtypeStruct((C2 * B2, H), jnp.float32)),
      mesh=mesh,
      scratch_types=[pltpu.VMEM((B01,), jnp.int32),
                     pltpu.VMEM((B01, H), jnp.float32),
                     pltpu.VMEM((B2,), jnp.int32),
                     pltpu.VMEM((B2, H), jnp.float32),
                     pltpu.SemaphoreType.DMA],
  )
  def _sc_gather(ns_hbm, v01_hbm, v2_hbm, x01_hbm, x2_hbm,
                 idx_v, rows_v, idx2_v, rows2_v, sem):
    wid = lax.axis_index("s") * NC + lax.axis_index("c")

    def body01(k, carry):
      c = wid + NW * k

      @pl.when(c < C01)
      def _():
        pltpu.sync_copy(v01_hbm.at[pl.ds(c * B01, B01)], idx_v)
        pltpu.async_copy(ns_hbm.at[idx_v], rows_v, sem).wait()
        pltpu.sync_copy(rows_v, x01_hbm.at[pl.ds(c * B01, B01)])
      return carry

    lax.fori_loop(0, K01, body01, 0)

    def body2(k, carry):
      c = wid + NW * k

      @pl.when(c < C2)
      def _():
        pltpu.sync_copy(v2_hbm.at[pl.ds(c * B2, B2)], idx2_v)
        pltpu.async_copy(ns_hbm.at[idx2_v], rows2_v, sem).wait()
        pltpu.sync_copy(rows2_v, x2_hbm.at[pl.ds(c * B2, B2)])
      return carry

    lax.fori_loop(0, K2, body2, 0)

  return _sc_gather


@functools.cache
def _build_scatter():
  mesh = plsc.VectorSubcoreMesh(core_axis_name="c", subcore_axis_name="s",
                                num_cores=NC, num_subcores=NS)

  @functools.partial(
      pl.kernel,
      out_type=jax.ShapeDtypeStruct((NC, N, H), jnp.float32),
      mesh=mesh,
      scratch_types=[pltpu.VMEM((B01,), jnp.int32),
                     pltpu.VMEM((B01, H), jnp.float32),
                     pltpu.VMEM((B2,), jnp.int32),
                     pltpu.VMEM((B2, H), jnp.float32),
                     pltpu.VMEM_SHARED((N, H), jnp.float32),
                     pltpu.SemaphoreType.DMA],
  )
  def _sc_scatter(e01_hbm, e2_hbm, v01_hbm, v2_hbm, out_hbm,
                  idx_v, buf_v, idx2_v, buf2_v, acc, sem):
    cid = lax.axis_index("c")
    sid = lax.axis_index("s")
    wid = sid * NC + cid

    def zrow(r, carry):
      for j in range(H // 16):
        buf2_v[r, pl.ds(j * 16, 16)] = jnp.zeros((16,), jnp.float32)
      return carry

    lax.fori_loop(0, ZB, zrow, 0)

    def zchunk(k, carry):
      c = sid + NS * k

      @pl.when(c < CZ)
      def _():
        pltpu.sync_copy(buf2_v, acc.at[pl.ds(c * ZB, ZB)])
      return carry

    lax.fori_loop(0, KZ, zchunk, 0)
    plsc.subcore_barrier()

    def body01(k, carry):
      c = wid + NW * k

      @pl.when(c < C01)
      def _():
        pltpu.sync_copy(v01_hbm.at[pl.ds(c * B01, B01)], idx_v)
        pltpu.sync_copy(e01_hbm.at[pl.ds(c * B01, B01)], buf_v)
        pltpu.sync_copy(buf_v, acc.at[idx_v], add=True)
      return carry

    lax.fori_loop(0, K01, body01, 0)

    def body2(k, carry):
      c = wid + NW * k

      @pl.when(c < C2)
      def _():
        pltpu.sync_copy(v2_hbm.at[pl.ds(c * B2, B2)], idx2_v)
        pltpu.sync_copy(e2_hbm.at[pl.ds(c * B2, B2)], buf2_v)
        pltpu.sync_copy(buf2_v, acc.at[idx2_v], add=True)
      return carry

    lax.fori_loop(0, K2, body2, 0)
    plsc.subcore_barrier()

    def ochunk(k, carry):
      c = sid + NS * k

      @pl.when(c < CZ)
      def _():
        pltpu.sync_copy(acc.at[pl.ds(c * ZB, ZB)],
                        out_hbm.at[cid, pl.ds(c * ZB, ZB)])
      return carry

    lax.fori_loop(0, KZ, ochunk, 0)

  return _sc_scatter


def _prep_body(has_extra, ns_ref, wgeT_ref, wex0_ref, wexCT_ref, b1_ref,
               c0_ref):
    x = ns_ref[...]
    off = jnp.max(x, axis=0, keepdims=True)
    s = jnp.sum(jnp.exp((x - off) * 8.0), axis=0, keepdims=True)
    ge = 0.125 * jnp.log(s) + off
    c0 = jnp.dot(ge, wgeT_ref[...], preferred_element_type=jnp.float32)
    c0 = c0 + b1_ref[...]
    if has_extra:
        counts = jnp.sum(x[:, :T], axis=0, keepdims=True)
        c0 = c0 + (N / float(MAXOBJ)) * wex0_ref[...]
        c0 = c0 + jnp.dot(counts * (1.0 / N), wexCT_ref[...],
                          preferred_element_type=jnp.float32)
    c0_ref[...] = c0


def _prep(ns, wgeT, wex0, wexCT, b1, has_extra):
    return pl.pallas_call(
        functools.partial(_prep_body, has_extra),
        out_shape=jax.ShapeDtypeStruct((1, 2 * H), jnp.float32),
    )(ns, wgeT, wex0, wexCT, b1)


def _mlp01_body(x_ref, w1T_ref, b1_ref, w2T_ref, b2_ref, o_ref):
    x = x_ref[0]
    h = jnp.maximum(
        jnp.dot(x, w1T_ref[0], preferred_element_type=jnp.float32)
        + b1_ref[0], 0.0)
    o = jnp.dot(h, w2T_ref[0], preferred_element_type=jnp.float32) + b2_ref[0]
    o_ref[0] = jnp.exp(8.0 * o)


def _mlp01(x01, w1T, b1, w2T, b2, bf):
    nb = 80000 // bf
    d = 2 * H
    return pl.pallas_call(
        _mlp01_body,
        grid=(2, nb),
        in_specs=[pl.BlockSpec((1, bf, d), lambda r, i: (r, i, 0)),
                  pl.BlockSpec((1, d, d), lambda r, i: (r, 0, 0)),
                  pl.BlockSpec((1, 1, d), lambda r, i: (r, 0, 0)),
                  pl.BlockSpec((1, d, d), lambda r, i: (r, 0, 0)),
                  pl.BlockSpec((1, 1, d), lambda r, i: (r, 0, 0))],
        out_specs=pl.BlockSpec((1, bf, d), lambda r, i: (r, i, 0)),
        out_shape=jax.ShapeDtypeStruct((2, 80000, d), jnp.float32),
    )(x01, w1T, b1, w2T, b2)


def _mlp2_body(x_ref, w1T_ref, b1_ref, w2T_ref, b2_ref, o_ref):
    x = x_ref[...]
    h = jnp.maximum(
        jnp.dot(x, w1T_ref[...], preferred_element_type=jnp.float32)
        + b1_ref[...], 0.0)
    o = jnp.dot(h, w2T_ref[...], preferred_element_type=jnp.float32)
    o_ref[...] = jnp.exp(8.0 * (o + b2_ref[...]))


def _mlp2(x2, w1T, b1, w2T, b2, bf):
    nb = 10000 // bf
    return pl.pallas_call(
        _mlp2_body,
        grid=(nb,),
        in_specs=[pl.BlockSpec((bf, H), lambda i: (i, 0)),
                  pl.BlockSpec((H, H), lambda i: (0, 0)),
                  pl.BlockSpec((1, H), lambda i: (0, 0)),
                  pl.BlockSpec((H, H), lambda i: (0, 0)),
                  pl.BlockSpec((1, H), lambda i: (0, 0))],
        out_specs=pl.BlockSpec((bf, H), lambda i: (i, 0)),
        out_shape=jax.ShapeDtypeStruct((10000, H), jnp.float32),
    )(x2, w1T, b1, w2T, b2)


def _upd_body(p_ref, ns_ref, c0_ref, w1mT_ref, w1nsT_ref, w2T_ref, b2_ref,
              o_ref):
    p = p_ref[0] + p_ref[1]
    m = 0.125 * jnp.log(p + 1e-16)
    h = jnp.maximum(
        jnp.dot(m, w1mT_ref[...], preferred_element_type=jnp.float32)
        + jnp.dot(ns_ref[...], w1nsT_ref[...],
                  preferred_element_type=jnp.float32)
        + c0_ref[...], 0.0)
    o_ref[...] = jnp.dot(h, w2T_ref[...],
                         preferred_element_type=jnp.float32) + b2_ref[...]


def _upd(part, ns, c0, w1mT, w1nsT, w2T, b2, bf):
    nb = N // bf
    return pl.pallas_call(
        _upd_body,
        grid=(nb,),
        in_specs=[pl.BlockSpec((NC, bf, H), lambda i: (0, i, 0)),
                  pl.BlockSpec((bf, H), lambda i: (i, 0)),
                  pl.BlockSpec((1, 2 * H), lambda i: (0, 0)),
                  pl.BlockSpec((H, 2 * H), lambda i: (0, 0)),
                  pl.BlockSpec((H, 2 * H), lambda i: (0, 0)),
                  pl.BlockSpec((2 * H, H), lambda i: (0, 0)),
                  pl.BlockSpec((1, H), lambda i: (0, 0))],
        out_specs=pl.BlockSpec((bf, H), lambda i: (i, 0)),
        out_shape=jax.ShapeDtypeStruct((N, H), jnp.float32),
    )(part, ns, c0, w1mT, w1nsT, w2T, b2)


def _gather_fn(ns, v01, v2):
    return _build_gather()(ns, v01, v2)


def _scatter_fn(e01, e2, v01, v2):
    return _build_scatter()(e01, e2, v01, v2)


def kernel(type_ids, rel0_values, rel1_values, rel2_values, init_random,
           r0_W1, r0_b1, r0_W2, r0_b2, r1_W1, r1_b1, r1_W2, r1_b2,
           r2_W1, r2_b1, r2_W2, r2_b2,
           u_W1, u_b1, u_W2, u_b2, v_W1, v_b1, v_W2, v_b2):
    f32 = jnp.float32
    ns = jnp.concatenate(
        [jax.nn.one_hot(type_ids, T, dtype=f32), init_random], axis=1)
    v01 = jnp.concatenate([rel0_values, rel1_values]).astype(jnp.int32)
    v2 = rel2_values.astype(jnp.int32)

    rW1T = jnp.stack([r0_W1.T, r1_W1.T])
    rb1 = jnp.stack([r0_b1, r1_b1]).reshape(2, 1, 2 * H)
    rW2T = jnp.stack([r0_W2.T, r1_W2.T])
    rb2 = jnp.stack([r0_b2, r1_b2]).reshape(2, 1, 2 * H)
    w2_1T = r2_W1.T
    w2_2T = r2_W2.T
    b2_1 = r2_b1.reshape(1, H)
    b2_2 = r2_b2.reshape(1, H)

    E = T + 1
    v_geT = v_W1[:, E:E + H].T
    v_mT = v_W1[:, E + H:E + 2 * H].T
    v_nsT = v_W1[:, E + 2 * H:].T
    v_ex0 = v_W1[:, 0:1].T
    v_exCT = v_W1[:, 1:E].T
    u_geT = u_W1[:, :H].T
    u_mT = u_W1[:, H:2 * H].T
    u_nsT = u_W1[:, 2 * H:].T
    zpad = jnp.zeros((T, 2 * H), f32)
    zpad1 = jnp.zeros((1, 2 * H), f32)

    for it in range(2):
        if it == 0:
            c0 = _prep(ns, v_geT, v_ex0, v_exCT, v_b1.reshape(1, -1), True)
            w1mT, w1nsT = v_mT, v_nsT
            w2T, b2 = v_W2.T, v_b2.reshape(1, H)
        else:
            c0 = _prep(ns, u_geT, zpad1, zpad, u_b1.reshape(1, -1), False)
            w1mT, w1nsT = u_mT, u_nsT
            w2T, b2 = u_W2.T, u_b2.reshape(1, H)

        x01, x2 = _gather_fn(ns, v01, v2)
        e01 = _mlp01(x01.reshape(2, 80000, 2 * H), rW1T, rb1, rW2T, rb2, 1000)
        e2 = _mlp2(x2, w2_1T, b2_1, w2_2T, b2_2, 1000)
        part = _scatter_fn(e01.reshape(320000, H), e2, v01, v2)
        ns = _upd(part, ns, c0, w1mT, w1nsT, w2T, b2, 1000)
    return ns


# speedup vs baseline: 2.6189x; 1.2529x over previous
import functools

import jax
import jax.numpy as jnp
from jax import lax
from jax.experimental import pallas as pl
from jax.experimental.pallas import tpu as pltpu
from jax.experimental.pallas import tpu_sc as plsc

N = 10000
T = 8
H = 128
MAXOBJ = 20000

NC = 2
NS = 16
NW = NC * NS

B = 80
C01 = 320000 // B
C2 = 10000 // B
T01 = C01 // NW
T2 = -(-C2 // NW)
TT = T01 + T2
ZB = B
CZ = N // ZB
KZ = -(-CZ // NS)

@functools.cache
def _build_gather():
  mesh = plsc.VectorSubcoreMesh(core_axis_name="c", subcore_axis_name="s",
                                num_cores=NC, num_subcores=NS)

  @functools.partial(
      pl.kernel,
      out_type=(jax.ShapeDtypeStruct((C01 * B, H), jnp.float32),
                jax.ShapeDtypeStruct((C2 * B, H), jnp.float32)),
      mesh=mesh,
      scratch_types=[pltpu.VMEM((B,), jnp.int32),
                     pltpu.VMEM((B,), jnp.int32),
                     pltpu.VMEM((B,), jnp.int32),
                     pltpu.VMEM((B, H), jnp.float32),
                     pltpu.VMEM((B, H), jnp.float32),
                     pltpu.VMEM((B, H), jnp.float32)]
                    + [pltpu.SemaphoreType.DMA] * 9,
  )
  def _sc_gather(ns_hbm, v01_hbm, v2_hbm, x01_hbm, x2_hbm,
                 ib0, ib1, ib2, rb0, rb1, rb2,
                 si0, si1, si2, sg0, sg1, sg2, ss0, ss1, ss2):
    wid = lax.axis_index("s") * NC + lax.axis_index("c")
    idxb, rowb = (ib0, ib1, ib2), (rb0, rb1, rb2)
    si, sg, ss = (si0, si1, si2), (sg0, sg1, sg2), (ss0, ss1, ss2)

    def src_dst(t):
      if t < T01:
        return v01_hbm, x01_hbm, wid + NW * t
      return v2_hbm, x2_hbm, wid + NW * (t - T01)

    def iload(t):
      v, _, c = src_dst(t)
      return pltpu.async_copy(v.at[pl.ds(c * B, B)], idxb[t % 3], si[t % 3])

    def gath(t):
      m = t % 3
      return pltpu.async_copy(ns_hbm.at[idxb[m]], rowb[m], sg[m])

    def store(t):
      _, x, c = src_dst(t)
      m = t % 3
      return pltpu.async_copy(rowb[m], x.at[pl.ds(c * B, B)], ss[m])

    NK = TT - 1
    d = {}
    d["i", 0] = iload(0)
    d["i", 1] = iload(1)
    for t in range(NK):
      d["i", t].wait()
      if t >= 3:
        d["s", t - 3].wait()
      d["g", t] = gath(t)
      if t >= 1:
        d["g", t - 1].wait()
        d["s", t - 1] = store(t - 1)
      if t + 2 < NK:
        d["i", t + 2] = iload(t + 2)
    d["g", NK - 1].wait()
    d["s", NK - 1] = store(NK - 1)
    d["s", NK - 3].wait()
    d["s", NK - 2].wait()
    d["s", NK - 1].wait()

    c2 = wid + NW * (T2 - 1)

    @pl.when(c2 < C2)
    def _():
      pltpu.sync_copy(v2_hbm.at[pl.ds(c2 * B, B)], ib0)
      pltpu.async_copy(ns_hbm.at[ib0], rb0, si0).wait()
      pltpu.sync_copy(rb0, x2_hbm.at[pl.ds(c2 * B, B)])

  return _sc_gather


@functools.cache
def _build_scatter():
  mesh = plsc.VectorSubcoreMesh(core_axis_name="c", subcore_axis_name="s",
                                num_cores=NC, num_subcores=NS)

  @functools.partial(
      pl.kernel,
      out_type=jax.ShapeDtypeStruct((NC, N, H), jnp.float32),
      mesh=mesh,
      scratch_types=[pltpu.VMEM((B,), jnp.int32),
                     pltpu.VMEM((B,), jnp.int32),
                     pltpu.VMEM((B,), jnp.int32),
                     pltpu.VMEM((B, H), jnp.float32),
                     pltpu.VMEM((B, H), jnp.float32),
                     pltpu.VMEM((B, H), jnp.float32),
                     pltpu.VMEM_SHARED((N, H), jnp.float32)]
                    + [pltpu.SemaphoreType.DMA] * 9,
  )
  def _sc_scatter(e01_hbm, e2_hbm, v01_hbm, v2_hbm, out_hbm,
                  ib0, ib1, ib2, bb0, bb1, bb2, acc,
                  si0, si1, si2, sl0, sl1, sl2, sa0, sa1, sa2):
    cid = lax.axis_index("c")
    sid = lax.axis_index("s")
    wid = sid * NC + cid
    idxb, bufb = (ib0, ib1, ib2), (bb0, bb1, bb2)
    si, sl, sa = (si0, si1, si2), (sl0, sl1, sl2), (sa0, sa1, sa2)

    def zrow(r, carry):
      for j in range(H // 16):
        bb0[r, pl.ds(j * 16, 16)] = jnp.zeros((16,), jnp.float32)
      return carry

    lax.fori_loop(0, ZB, zrow, 0)

    def zchunk(k, carry):
      c = sid + NS * k

      @pl.when(c < CZ)
      def _():
        pltpu.sync_copy(bb0, acc.at[pl.ds(c * ZB, ZB)])
      return carry

    lax.fori_loop(0, KZ, zchunk, 0)
    plsc.subcore_barrier()

    def srcs(t):
      if t < T01:
        return e01_hbm, v01_hbm, wid + NW * t
      return e2_hbm, v2_hbm, wid + NW * (t - T01)

    def iload(t):
      _, v, c = srcs(t)
      m = t % 3
      return pltpu.async_copy(v.at[pl.ds(c * B, B)], idxb[m], si[m])

    def dload(t):
      e, _, c = srcs(t)
      m = t % 3
      return pltpu.async_copy(e.at[pl.ds(c * B, B)], bufb[m], sl[m])

    def addc(t):
      m = t % 3
      return pltpu.async_copy(bufb[m], acc.at[idxb[m]], sa[m], add=True)

    NK = TT - 1
    d = {}
    d["i", 0] = iload(0)
    d["l", 0] = dload(0)
    d["i", 1] = iload(1)
    d["l", 1] = dload(1)
    for t in range(NK):
      d["i", t].wait()
      d["l", t].wait()
      d["a", t] = addc(t)
      if t + 2 < NK:
        if t >= 1:
          d["a", t - 1].wait()
        d["i", t + 2] = iload(t + 2)
        d["l", t + 2] = dload(t + 2)
    d["a", NK - 2].wait()
    d["a", NK - 1].wait()

    c2 = wid + NW * (T2 - 1)

    @pl.when(c2 < C2)
    def _():
      pltpu.sync_copy(v2_hbm.at[pl.ds(c2 * B, B)], ib0)
      pltpu.sync_copy(e2_hbm.at[pl.ds(c2 * B, B)], bb0)
      pltpu.sync_copy(bb0, acc.at[ib0], add=True)

    plsc.subcore_barrier()

    def ochunk(k, carry):
      c = sid + NS * k

      @pl.when(c < CZ)
      def _():
        pltpu.sync_copy(acc.at[pl.ds(c * ZB, ZB)],
                        out_hbm.at[cid, pl.ds(c * ZB, ZB)])
      return carry

    lax.fori_loop(0, KZ, ochunk, 0)

  return _sc_scatter


def _prep_body(has_extra, ns_ref, wgeT_ref, wex0_ref, wexCT_ref, b1_ref,
               c0_ref):
    x = ns_ref[...]
    off = jnp.max(x, axis=0, keepdims=True)
    s = jnp.sum(jnp.exp((x - off) * 8.0), axis=0, keepdims=True)
    ge = 0.125 * jnp.log(s) + off
    c0 = jnp.dot(ge, wgeT_ref[...], preferred_element_type=jnp.float32)
    c0 = c0 + b1_ref[...]
    if has_extra:
        counts = jnp.sum(x[:, :T], axis=0, keepdims=True)
        c0 = c0 + (N / float(MAXOBJ)) * wex0_ref[...]
        c0 = c0 + jnp.dot(counts * (1.0 / N), wexCT_ref[...],
                          preferred_element_type=jnp.float32)
    c0_ref[...] = c0


def _prep(ns, wgeT, wex0, wexCT, b1, has_extra):
    return pl.pallas_call(
        functools.partial(_prep_body, has_extra),
        out_shape=jax.ShapeDtypeStruct((1, 2 * H), jnp.float32),
    )(ns, wgeT, wex0, wexCT, b1)


def _mlp01_body(x_ref, w1T_ref, b1_ref, w2T_ref, b2_ref, o_ref):
    x = x_ref[0]
    h = jnp.maximum(
        jnp.dot(x, w1T_ref[0], preferred_element_type=jnp.float32)
        + b1_ref[0], 0.0)
    o = jnp.dot(h, w2T_ref[0], preferred_element_type=jnp.float32) + b2_ref[0]
    o_ref[0] = jnp.exp(8.0 * o)


def _mlp01(x01, w1T, b1, w2T, b2, bf):
    nb = 80000 // bf
    d = 2 * H
    return pl.pallas_call(
        _mlp01_body,
        grid=(2, nb),
        in_specs=[pl.BlockSpec((1, bf, d), lambda r, i: (r, i, 0)),
                  pl.BlockSpec((1, d, d), lambda r, i: (r, 0, 0)),
                  pl.BlockSpec((1, 1, d), lambda r, i: (r, 0, 0)),
                  pl.BlockSpec((1, d, d), lambda r, i: (r, 0, 0)),
                  pl.BlockSpec((1, 1, d), lambda r, i: (r, 0, 0))],
        out_specs=pl.BlockSpec((1, bf, d), lambda r, i: (r, i, 0)),
        out_shape=jax.ShapeDtypeStruct((2, 80000, d), jnp.float32),
    )(x01, w1T, b1, w2T, b2)


def _mlp2_body(x_ref, w1T_ref, b1_ref, w2T_ref, b2_ref, o_ref):
    x = x_ref[...]
    h = jnp.maximum(
        jnp.dot(x, w1T_ref[...], preferred_element_type=jnp.float32)
        + b1_ref[...], 0.0)
    o = jnp.dot(h, w2T_ref[...], preferred_element_type=jnp.float32)
    o_ref[...] = jnp.exp(8.0 * (o + b2_ref[...]))


def _mlp2(x2, w1T, b1, w2T, b2, bf):
    nb = 10000 // bf
    return pl.pallas_call(
        _mlp2_body,
        grid=(nb,),
        in_specs=[pl.BlockSpec((bf, H), lambda i: (i, 0)),
                  pl.BlockSpec((H, H), lambda i: (0, 0)),
                  pl.BlockSpec((1, H), lambda i: (0, 0)),
                  pl.BlockSpec((H, H), lambda i: (0, 0)),
                  pl.BlockSpec((1, H), lambda i: (0, 0))],
        out_specs=pl.BlockSpec((bf, H), lambda i: (i, 0)),
        out_shape=jax.ShapeDtypeStruct((10000, H), jnp.float32),
    )(x2, w1T, b1, w2T, b2)


def _upd_body(p_ref, ns_ref, c0_ref, w1mT_ref, w1nsT_ref, w2T_ref, b2_ref,
              o_ref):
    p = p_ref[0] + p_ref[1]
    m = 0.125 * jnp.log(p + 1e-16)
    h = jnp.maximum(
        jnp.dot(m, w1mT_ref[...], preferred_element_type=jnp.float32)
        + jnp.dot(ns_ref[...], w1nsT_ref[...],
                  preferred_element_type=jnp.float32)
        + c0_ref[...], 0.0)
    o_ref[...] = jnp.dot(h, w2T_ref[...],
                         preferred_element_type=jnp.float32) + b2_ref[...]


def _upd(part, ns, c0, w1mT, w1nsT, w2T, b2, bf):
    nb = N // bf
    return pl.pallas_call(
        _upd_body,
        grid=(nb,),
        in_specs=[pl.BlockSpec((NC, bf, H), lambda i: (0, i, 0)),
                  pl.BlockSpec((bf, H), lambda i: (i, 0)),
                  pl.BlockSpec((1, 2 * H), lambda i: (0, 0)),
                  pl.BlockSpec((H, 2 * H), lambda i: (0, 0)),
                  pl.BlockSpec((H, 2 * H), lambda i: (0, 0)),
                  pl.BlockSpec((2 * H, H), lambda i: (0, 0)),
                  pl.BlockSpec((1, H), lambda i: (0, 0))],
        out_specs=pl.BlockSpec((bf, H), lambda i: (i, 0)),
        out_shape=jax.ShapeDtypeStruct((N, H), jnp.float32),
    )(part, ns, c0, w1mT, w1nsT, w2T, b2)


def _gather_fn(ns, v01, v2):
    return _build_gather()(ns, v01, v2)


def _scatter_fn(e01, e2, v01, v2):
    return _build_scatter()(e01, e2, v01, v2)


def kernel(type_ids, rel0_values, rel1_values, rel2_values, init_random,
           r0_W1, r0_b1, r0_W2, r0_b2, r1_W1, r1_b1, r1_W2, r1_b2,
           r2_W1, r2_b1, r2_W2, r2_b2,
           u_W1, u_b1, u_W2, u_b2, v_W1, v_b1, v_W2, v_b2):
    f32 = jnp.float32
    ns = jnp.concatenate(
        [jax.nn.one_hot(type_ids, T, dtype=f32), init_random], axis=1)
    v01 = jnp.concatenate([rel0_values, rel1_values]).astype(jnp.int32)
    v2 = rel2_values.astype(jnp.int32)

    rW1T = jnp.stack([r0_W1.T, r1_W1.T])
    rb1 = jnp.stack([r0_b1, r1_b1]).reshape(2, 1, 2 * H)
    rW2T = jnp.stack([r0_W2.T, r1_W2.T])
    rb2 = jnp.stack([r0_b2, r1_b2]).reshape(2, 1, 2 * H)
    w2_1T = r2_W1.T
    w2_2T = r2_W2.T
    b2_1 = r2_b1.reshape(1, H)
    b2_2 = r2_b2.reshape(1, H)

    E = T + 1
    v_geT = v_W1[:, E:E + H].T
    v_mT = v_W1[:, E + H:E + 2 * H].T
    v_nsT = v_W1[:, E + 2 * H:].T
    v_ex0 = v_W1[:, 0:1].T
    v_exCT = v_W1[:, 1:E].T
    u_geT = u_W1[:, :H].T
    u_mT = u_W1[:, H:2 * H].T
    u_nsT = u_W1[:, 2 * H:].T
    zpad = jnp.zeros((T, 2 * H), f32)
    zpad1 = jnp.zeros((1, 2 * H), f32)

    for it in range(2):
        if it == 0:
            c0 = _prep(ns, v_geT, v_ex0, v_exCT, v_b1.reshape(1, -1), True)
            w1mT, w1nsT = v_mT, v_nsT
            w2T, b2 = v_W2.T, v_b2.reshape(1, H)
        else:
            c0 = _prep(ns, u_geT, zpad1, zpad, u_b1.reshape(1, -1), False)
            w1mT, w1nsT = u_mT, u_nsT
            w2T, b2 = u_W2.T, u_b2.reshape(1, H)

        x01, x2 = _gather_fn(ns, v01, v2)
        e01 = _mlp01(x01.reshape(2, 80000, 2 * H), rW1T, rb1, rW2T, rb2, 1000)
        e2 = _mlp2(x2, w2_1T, b2_1, w2_2T, b2_2, 1000)
        part = _scatter_fn(e01.reshape(320000, H), e2, v01, v2)
        ns = _upd(part, ns, c0, w1mT, w1nsT, w2T, b2, 1000)
    return ns


# speedup vs baseline: 4.1928x; 1.6010x over previous
import functools

import jax
import jax.numpy as jnp
from jax import lax
from jax.experimental import pallas as pl
from jax.experimental.pallas import tpu as pltpu
from jax.experimental.pallas import tpu_sc as plsc

N = 10000
T = 8
H = 128
MAXOBJ = 20000

NC = 2
NS = 16
NW = NC * NS

B = 80
CA = 160000 // B
C2 = 10000 // B
TAU = CA // NW
T2U = C2 // NW
_STEPS = ([("a", t) for t in range(TAU)] + [("b", t) for t in range(TAU)]
          + [("2", t) for t in range(T2U)])
_TAILS = [("a", TAU, CA), ("b", TAU, CA), ("2", T2U, C2)]
ZB = B
CZ = N // ZB
KZ = -(-CZ // NS)

@functools.cache
def _build_gather():
  mesh = plsc.VectorSubcoreMesh(core_axis_name="c", subcore_axis_name="s",
                                num_cores=NC, num_subcores=NS)

  @functools.partial(
      pl.kernel,
      out_type=(jax.ShapeDtypeStruct((CA * B, H), jnp.float32),
                jax.ShapeDtypeStruct((CA * B, H), jnp.float32),
                jax.ShapeDtypeStruct((C2 * B, H), jnp.float32)),
      mesh=mesh,
      scratch_types=[pltpu.VMEM((B,), jnp.int32),
                     pltpu.VMEM((B,), jnp.int32),
                     pltpu.VMEM((B,), jnp.int32),
                     pltpu.VMEM((B, H), jnp.float32),
                     pltpu.VMEM((B, H), jnp.float32),
                     pltpu.VMEM((B, H), jnp.float32)]
                    + [pltpu.SemaphoreType.DMA] * 9,
  )
  def _sc_gather(ns_hbm, va_hbm, vb_hbm, v2_hbm, xa_hbm, xb_hbm, x2_hbm,
                 ib0, ib1, ib2, rb0, rb1, rb2,
                 si0, si1, si2, sg0, sg1, sg2, ss0, ss1, ss2):
    wid = lax.axis_index("s") * NC + lax.axis_index("c")
    idxb, rowb = (ib0, ib1, ib2), (rb0, rb1, rb2)
    si, sg, ss = (si0, si1, si2), (sg0, sg1, sg2), (ss0, ss1, ss2)
    refs = {"a": (va_hbm, xa_hbm), "b": (vb_hbm, xb_hbm),
            "2": (v2_hbm, x2_hbm)}

    def src_dst(j):
      s, t = _STEPS[j]
      v, x = refs[s]
      return v, x, wid + NW * t

    def iload(j):
      v, _, c = src_dst(j)
      return pltpu.async_copy(v.at[pl.ds(c * B, B)], idxb[j % 3], si[j % 3])

    def gath(j):
      m = j % 3
      return pltpu.async_copy(ns_hbm.at[idxb[m]], rowb[m], sg[m])

    def store(j):
      _, x, c = src_dst(j)
      m = j % 3
      return pltpu.async_copy(rowb[m], x.at[pl.ds(c * B, B)], ss[m])

    NK = len(_STEPS)
    d = {}
    d["i", 0] = iload(0)
    d["i", 1] = iload(1)
    for j in range(NK):
      d["i", j].wait()
      if j >= 3:
        d["s", j - 3].wait()
      d["g", j] = gath(j)
      if j >= 1:
        d["g", j - 1].wait()
        d["s", j - 1] = store(j - 1)
      if j + 2 < NK:
        d["i", j + 2] = iload(j + 2)
    d["g", NK - 1].wait()
    d["s", NK - 1] = store(NK - 1)
    d["s", NK - 3].wait()
    d["s", NK - 2].wait()
    d["s", NK - 1].wait()

    for s, t, cmax in _TAILS:
      v, x = refs[s]
      c = wid + NW * t

      @pl.when(c < cmax)
      def _(v=v, x=x, c=c):
        pltpu.sync_copy(v.at[pl.ds(c * B, B)], ib0)
        pltpu.async_copy(ns_hbm.at[ib0], rb0, si0).wait()
        pltpu.sync_copy(rb0, x.at[pl.ds(c * B, B)])

  return _sc_gather


@functools.cache
def _build_scatter():
  mesh = plsc.VectorSubcoreMesh(core_axis_name="c", subcore_axis_name="s",
                                num_cores=NC, num_subcores=NS)

  @functools.partial(
      pl.kernel,
      out_type=jax.ShapeDtypeStruct((NC, N, H), jnp.float32),
      mesh=mesh,
      scratch_types=[pltpu.VMEM((B,), jnp.int32),
                     pltpu.VMEM((B,), jnp.int32),
                     pltpu.VMEM((B,), jnp.int32),
                     pltpu.VMEM((B, H), jnp.float32),
                     pltpu.VMEM((B, H), jnp.float32),
                     pltpu.VMEM((B, H), jnp.float32),
                     pltpu.VMEM_SHARED((N, H), jnp.float32)]
                    + [pltpu.SemaphoreType.DMA] * 9,
  )
  def _sc_scatter(ea_hbm, eb_hbm, e2_hbm, va_hbm, vb_hbm, v2_hbm, out_hbm,
                  ib0, ib1, ib2, bb0, bb1, bb2, acc,
                  si0, si1, si2, sl0, sl1, sl2, sa0, sa1, sa2):
    cid = lax.axis_index("c")
    sid = lax.axis_index("s")
    wid = sid * NC + cid
    idxb, bufb = (ib0, ib1, ib2), (bb0, bb1, bb2)
    si, sl, sa = (si0, si1, si2), (sl0, sl1, sl2), (sa0, sa1, sa2)
    refs = {"a": (ea_hbm, va_hbm), "b": (eb_hbm, vb_hbm),
            "2": (e2_hbm, v2_hbm)}

    def zrow(r, carry):
      for j in range(H // 16):
        bb0[r, pl.ds(j * 16, 16)] = jnp.zeros((16,), jnp.float32)
      return carry

    lax.fori_loop(0, ZB, zrow, 0)

    def zchunk(k, carry):
      c = sid + NS * k

      @pl.when(c < CZ)
      def _():
        pltpu.sync_copy(bb0, acc.at[pl.ds(c * ZB, ZB)])
      return carry

    lax.fori_loop(0, KZ, zchunk, 0)
    plsc.subcore_barrier()

    def srcs(j):
      s, t = _STEPS[j]
      e, v = refs[s]
      return e, v, wid + NW * t

    def iload(j):
      _, v, c = srcs(j)
      m = j % 3
      return pltpu.async_copy(v.at[pl.ds(c * B, B)], idxb[m], si[m])

    def dload(j):
      e, _, c = srcs(j)
      m = j % 3
      return pltpu.async_copy(e.at[pl.ds(c * B, B)], bufb[m], sl[m])

    def addc(j):
      m = j % 3
      return pltpu.async_copy(bufb[m], acc.at[idxb[m]], sa[m], add=True)

    NK = len(_STEPS)
    d = {}
    d["i", 0] = iload(0)
    d["l", 0] = dload(0)
    d["i", 1] = iload(1)
    d["l", 1] = dload(1)
    for j in range(NK):
      d["i", j].wait()
      d["l", j].wait()
      d["a", j] = addc(j)
      if j >= 1:
        d["a", j - 1].wait()
      if j + 2 < NK:
        d["i", j + 2] = iload(j + 2)
        d["l", j + 2] = dload(j + 2)
    d["a", NK - 1].wait()

    for s, t, cmax in _TAILS:
      e, v = refs[s]
      c = wid + NW * t

      @pl.when(c < cmax)
      def _(e=e, v=v, c=c):
        pltpu.sync_copy(v.at[pl.ds(c * B, B)], ib0)
        pltpu.sync_copy(e.at[pl.ds(c * B, B)], bb0)
        pltpu.sync_copy(bb0, acc.at[ib0], add=True)

    plsc.subcore_barrier()

    def ochunk(k, carry):
      c = sid + NS * k

      @pl.when(c < CZ)
      def _():
        pltpu.sync_copy(acc.at[pl.ds(c * ZB, ZB)],
                        out_hbm.at[cid, pl.ds(c * ZB, ZB)])
      return carry

    lax.fori_loop(0, KZ, ochunk, 0)

  return _sc_scatter


def _prep_body(has_extra, ns_ref, wgeT_ref, wex0_ref, wexCT_ref, b1_ref,
               c0_ref):
    x = ns_ref[...]
    off = jnp.max(x, axis=0, keepdims=True)
    s = jnp.sum(jnp.exp((x - off) * 8.0), axis=0, keepdims=True)
    ge = 0.125 * jnp.log(s) + off
    c0 = jnp.dot(ge, wgeT_ref[...], preferred_element_type=jnp.float32)
    c0 = c0 + b1_ref[...]
    if has_extra:
        counts = jnp.sum(x[:, :T], axis=0, keepdims=True)
        c0 = c0 + (N / float(MAXOBJ)) * wex0_ref[...]
        c0 = c0 + jnp.dot(counts * (1.0 / N), wexCT_ref[...],
                          preferred_element_type=jnp.float32)
    c0_ref[...] = c0


def _prep(ns, wgeT, wex0, wexCT, b1, has_extra):
    return pl.pallas_call(
        functools.partial(_prep_body, has_extra),
        out_shape=jax.ShapeDtypeStruct((1, 2 * H), jnp.float32),
    )(ns, wgeT, wex0, wexCT, b1)


def _mlp01_body(xa_ref, xb_ref, w1T_ref, b1_ref, w2T_ref, b2_ref,
                ea_ref, eb_ref):
    x = jnp.concatenate([xa_ref[0], xb_ref[0]], axis=1)
    h = jnp.maximum(
        jnp.dot(x, w1T_ref[0], preferred_element_type=jnp.float32)
        + b1_ref[0], 0.0)
    o = jnp.dot(h, w2T_ref[0], preferred_element_type=jnp.float32) + b2_ref[0]
    e = jnp.exp(8.0 * o)
    ea_ref[0] = e[:, :H]
    eb_ref[0] = e[:, H:]


def _mlp01(xa, xb, w1T, b1, w2T, b2, bf):
    nb = 80000 // bf
    d = 2 * H
    espec = jax.ShapeDtypeStruct((2, 80000, H), jnp.float32)
    return pl.pallas_call(
        _mlp01_body,
        grid=(2, nb),
        in_specs=[pl.BlockSpec((1, bf, H), lambda r, i: (r, i, 0)),
                  pl.BlockSpec((1, bf, H), lambda r, i: (r, i, 0)),
                  pl.BlockSpec((1, d, d), lambda r, i: (r, 0, 0)),
                  pl.BlockSpec((1, 1, d), lambda r, i: (r, 0, 0)),
                  pl.BlockSpec((1, d, d), lambda r, i: (r, 0, 0)),
                  pl.BlockSpec((1, 1, d), lambda r, i: (r, 0, 0))],
        out_specs=[pl.BlockSpec((1, bf, H), lambda r, i: (r, i, 0)),
                   pl.BlockSpec((1, bf, H), lambda r, i: (r, i, 0))],
        out_shape=(espec, espec),
    )(xa, xb, w1T, b1, w2T, b2)


def _mlp2_body(x_ref, w1T_ref, b1_ref, w2T_ref, b2_ref, o_ref):
    x = x_ref[...]
    h = jnp.maximum(
        jnp.dot(x, w1T_ref[...], preferred_element_type=jnp.float32)
        + b1_ref[...], 0.0)
    o = jnp.dot(h, w2T_ref[...], preferred_element_type=jnp.float32)
    o_ref[...] = jnp.exp(8.0 * (o + b2_ref[...]))


def _mlp2(x2, w1T, b1, w2T, b2, bf):
    nb = 10000 // bf
    return pl.pallas_call(
        _mlp2_body,
        grid=(nb,),
        in_specs=[pl.BlockSpec((bf, H), lambda i: (i, 0)),
                  pl.BlockSpec((H, H), lambda i: (0, 0)),
                  pl.BlockSpec((1, H), lambda i: (0, 0)),
                  pl.BlockSpec((H, H), lambda i: (0, 0)),
                  pl.BlockSpec((1, H), lambda i: (0, 0))],
        out_specs=pl.BlockSpec((bf, H), lambda i: (i, 0)),
        out_shape=jax.ShapeDtypeStruct((10000, H), jnp.float32),
    )(x2, w1T, b1, w2T, b2)


def _upd_body(p_ref, ns_ref, c0_ref, w1mT_ref, w1nsT_ref, w2T_ref, b2_ref,
              o_ref):
    p = p_ref[0] + p_ref[1]
    m = 0.125 * jnp.log(p + 1e-16)
    h = jnp.maximum(
        jnp.dot(m, w1mT_ref[...], preferred_element_type=jnp.float32)
        + jnp.dot(ns_ref[...], w1nsT_ref[...],
                  preferred_element_type=jnp.float32)
        + c0_ref[...], 0.0)
    o_ref[...] = jnp.dot(h, w2T_ref[...],
                         preferred_element_type=jnp.float32) + b2_ref[...]


def _upd(part, ns, c0, w1mT, w1nsT, w2T, b2, bf):
    nb = N // bf
    return pl.pallas_call(
        _upd_body,
        grid=(nb,),
        in_specs=[pl.BlockSpec((NC, bf, H), lambda i: (0, i, 0)),
                  pl.BlockSpec((bf, H), lambda i: (i, 0)),
                  pl.BlockSpec((1, 2 * H), lambda i: (0, 0)),
                  pl.BlockSpec((H, 2 * H), lambda i: (0, 0)),
                  pl.BlockSpec((H, 2 * H), lambda i: (0, 0)),
                  pl.BlockSpec((2 * H, H), lambda i: (0, 0)),
                  pl.BlockSpec((1, H), lambda i: (0, 0))],
        out_specs=pl.BlockSpec((bf, H), lambda i: (i, 0)),
        out_shape=jax.ShapeDtypeStruct((N, H), jnp.float32),
    )(part, ns, c0, w1mT, w1nsT, w2T, b2)


def _gather_fn(ns, va, vb, v2):
    return _build_gather()(ns, va, vb, v2)


def _scatter_fn(ea, eb, e2, va, vb, v2):
    return _build_scatter()(ea, eb, e2, va, vb, v2)


def kernel(type_ids, rel0_values, rel1_values, rel2_values, init_random,
           r0_W1, r0_b1, r0_W2, r0_b2, r1_W1, r1_b1, r1_W2, r1_b2,
           r2_W1, r2_b1, r2_W2, r2_b2,
           u_W1, u_b1, u_W2, u_b2, v_W1, v_b1, v_W2, v_b2):
    f32 = jnp.float32
    ns = jnp.concatenate(
        [jax.nn.one_hot(type_ids, T, dtype=f32), init_random], axis=1)
    r0v = rel0_values.astype(jnp.int32)
    r1v = rel1_values.astype(jnp.int32)
    va = jnp.concatenate([r0v[0::2], r1v[0::2]])
    vb = jnp.concatenate([r0v[1::2], r1v[1::2]])
    v2 = rel2_values.astype(jnp.int32)

    rW1T = jnp.stack([r0_W1.T, r1_W1.T])
    rb1 = jnp.stack([r0_b1, r1_b1]).reshape(2, 1, 2 * H)
    rW2T = jnp.stack([r0_W2.T, r1_W2.T])
    rb2 = jnp.stack([r0_b2, r1_b2]).reshape(2, 1, 2 * H)
    w2_1T = r2_W1.T
    w2_2T = r2_W2.T
    b2_1 = r2_b1.reshape(1, H)
    b2_2 = r2_b2.reshape(1, H)

    E = T + 1
    v_geT = v_W1[:, E:E + H].T
    v_mT = v_W1[:, E + H:E + 2 * H].T
    v_nsT = v_W1[:, E + 2 * H:].T
    v_ex0 = v_W1[:, 0:1].T
    v_exCT = v_W1[:, 1:E].T
    u_geT = u_W1[:, :H].T
    u_mT = u_W1[:, H:2 * H].T
    u_nsT = u_W1[:, 2 * H:].T
    zpad = jnp.zeros((T, 2 * H), f32)
    zpad1 = jnp.zeros((1, 2 * H), f32)

    for it in range(2):
        if it == 0:
            c0 = _prep(ns, v_geT, v_ex0, v_exCT, v_b1.reshape(1, -1), True)
            w1mT, w1nsT = v_mT, v_nsT
            w2T, b2 = v_W2.T, v_b2.reshape(1, H)
        else:
            c0 = _prep(ns, u_geT, zpad1, zpad, u_b1.reshape(1, -1), False)
            w1mT, w1nsT = u_mT, u_nsT
            w2T, b2 = u_W2.T, u_b2.reshape(1, H)

        xa, xb, x2 = _gather_fn(ns, va, vb, v2)
        ea, eb = _mlp01(xa.reshape(2, 80000, H), xb.reshape(2, 80000, H),
                        rW1T, rb1, rW2T, rb2, 1000)
        e2 = _mlp2(x2, w2_1T, b2_1, w2_2T, b2_2, 1000)
        part = _scatter_fn(ea.reshape(160000, H), eb.reshape(160000, H), e2,
                           va, vb, v2)
        ns = _upd(part, ns, c0, w1mT, w1nsT, w2T, b2, 1000)
    return ns
